# baseline (speedup 1.0000x reference)
import functools

import jax
import jax.numpy as jnp
from jax.experimental import pallas as pl

EPS = 1e-12


def _embed_ln_kernel(x_ref, pos_ref, gamma_ref, beta_ref, out_ref):
    x = x_ref[...]
    pos = pos_ref[...]
    e = x + pos[None, :, :]
    mean = jnp.mean(e, axis=-1, keepdims=True)
    c = e - mean
    var = jnp.mean(c * c, axis=-1, keepdims=True)
    normed = c * jax.lax.rsqrt(var + EPS)
    out_ref[...] = normed * gamma_ref[...][None, None, :] + beta_ref[...][None, None, :]


@functools.partial(jax.jit, static_argnames=("blk",))
def _run(methylation_data, pos_table, gamma, beta, blk):
    B, S, H = methylation_data.shape
    grid = (S // blk, B)
    return pl.pallas_call(
        _embed_ln_kernel,
        grid=grid,
        in_specs=[
            pl.BlockSpec((1, blk, H), lambda j, b: (b, j, 0)),
            pl.BlockSpec((blk, H), lambda j, b: (j, 0)),
            pl.BlockSpec((H,), lambda j, b: (0,)),
            pl.BlockSpec((H,), lambda j, b: (0,)),
        ],
        out_specs=pl.BlockSpec((1, blk, H), lambda j, b: (b, j, 0)),
        out_shape=jax.ShapeDtypeStruct((B, S, H), methylation_data.dtype),
    )(methylation_data, pos_table, gamma, beta)


def kernel(methylation_data, pos_table, gamma, beta):
    S = methylation_data.shape[1]
    return _run(methylation_data, pos_table[:S], gamma, beta, blk=2048)


# speedup vs baseline: 1.0600x; 1.0600x over previous
import functools

import jax
import jax.numpy as jnp
from jax.experimental import pallas as pl
from jax.experimental.pallas import tpu as pltpu

EPS = 1e-12


_SUB = 256


def _embed_ln_kernel(x_ref, pos_ref, gamma_ref, beta_ref, out_ref):
    blk = x_ref.shape[1]
    gamma = gamma_ref[...][None, None, :]
    beta = beta_ref[...][None, None, :]
    for i in range(blk // _SUB):
        rows = pl.ds(i * _SUB, _SUB)
        e = x_ref[:, rows, :] + pos_ref[rows, :][None, :, :]
        mean = jnp.mean(e, axis=-1, keepdims=True)
        c = e - mean
        var = jnp.mean(c * c, axis=-1, keepdims=True)
        normed = c * jax.lax.rsqrt(var + EPS)
        out_ref[:, rows, :] = normed * gamma + beta


@functools.partial(jax.jit, static_argnames=("blk",))
def _run(methylation_data, pos_table, gamma, beta, blk):
    B, S, H = methylation_data.shape
    grid = (pl.cdiv(S, blk),)
    return pl.pallas_call(
        _embed_ln_kernel,
        grid=grid,
        in_specs=[
            pl.BlockSpec((B, blk, H), lambda j: (0, j, 0)),
            pl.BlockSpec((blk, H), lambda j: (j, 0)),
            pl.BlockSpec((H,), lambda j: (0,)),
            pl.BlockSpec((H,), lambda j: (0,)),
        ],
        out_specs=pl.BlockSpec((B, blk, H), lambda j: (0, j, 0)),
        out_shape=jax.ShapeDtypeStruct((B, S, H), methylation_data.dtype),
        compiler_params=pltpu.CompilerParams(
            vmem_limit_bytes=100 * 1024 * 1024,
        ),
    )(methylation_data, pos_table, gamma, beta)


def kernel(methylation_data, pos_table, gamma, beta):
    S = methylation_data.shape[1]
    return _run(methylation_data, pos_table[:S], gamma, beta, blk=1024)


# speedup vs baseline: 1.0643x; 1.0041x over previous
import functools

import jax
import jax.numpy as jnp
from jax.experimental import pallas as pl
from jax.experimental.pallas import tpu as pltpu

EPS = 1e-12


def _embed_ln_kernel(x_ref, pos_ref, gamma_ref, beta_ref, out_ref):
    e = x_ref[...] + pos_ref[...][None, :, :]
    mean = jnp.mean(e, axis=-1, keepdims=True)
    c = e - mean
    var = jnp.mean(c * c, axis=-1, keepdims=True)
    normed = c * jax.lax.rsqrt(var + EPS)
    out_ref[...] = normed * gamma_ref[...][None, None, :] + beta_ref[...][None, None, :]


@functools.partial(jax.jit, static_argnames=("blk",))
def _run(methylation_data, pos_table, gamma, beta, blk):
    B, S, H = methylation_data.shape
    grid = (pl.cdiv(S, blk),)
    return pl.pallas_call(
        _embed_ln_kernel,
        grid=grid,
        in_specs=[
            pl.BlockSpec((B, blk, H), lambda j: (0, j, 0)),
            pl.BlockSpec((blk, H), lambda j: (j, 0)),
            pl.BlockSpec((H,), lambda j: (0,)),
            pl.BlockSpec((H,), lambda j: (0,)),
        ],
        out_specs=pl.BlockSpec((B, blk, H), lambda j: (0, j, 0)),
        out_shape=jax.ShapeDtypeStruct((B, S, H), methylation_data.dtype),
        compiler_params=pltpu.CompilerParams(
            vmem_limit_bytes=100 * 1024 * 1024,
        ),
    )(methylation_data, pos_table, gamma, beta)


def kernel(methylation_data, pos_table, gamma, beta):
    S = methylation_data.shape[1]
    return _run(methylation_data, pos_table[:S], gamma, beta, blk=896)
